# baseline (speedup 1.0000x reference)
import functools

import jax
import jax.numpy as jnp
from jax import lax
from jax.experimental import pallas as pl
from jax.experimental.pallas import tpu as pltpu
from jax.experimental.pallas import tpu_sc as plsc

N = 10000
NP = 10240
D = 128
E = 320000
C = 128
NCHUNK = E // C
NT = 16
NW = 32
RPT = NP // NT
KBASE = NCHUNK // NW
KEXTRA = NCHUNK - KBASE * NW
KMAX = KBASE + 1
KW = 88
NBUF = 4
NLOOP = (NCHUNK + NW - 1) // NW

_MESH = plsc.VectorSubcoreMesh(core_axis_name="c", subcore_axis_name="s")


def _sc_eagg(dst_hbm, ef_hbm, z128_hbm, eagg_hbm, idx4, rows2, sems, acc):
    c = lax.axis_index("c")
    s = lax.axis_index("s")
    w = s * 2 + c
    sl = pl.ds(s * RPT, RPT)

    pltpu.sync_copy(z128_hbm.at[sl], acc.at[sl])
    plsc.subcore_barrier()

    @pl.loop(0, NLOOP + 1)
    def _(i):
        ka = i
        ca_ = w + ka * NW
        ba4 = lax.rem(ka, 4)
        ba2 = lax.rem(ka, 2)

        @pl.when(ca_ < NCHUNK)
        def _():
            @pl.when(ka >= 2)
            def _():
                pltpu.make_async_copy(
                    rows2.at[ba2], acc.at[idx4.at[ba4]], sems.at[2, ba2]).wait()
            pltpu.make_async_copy(
                dst_hbm.at[pl.ds(ca_ * C, C)], idx4.at[ba4],
                sems.at[0, ba4]).start()
            pltpu.make_async_copy(
                ef_hbm.at[pl.ds(ca_ * C, C)], rows2.at[ba2],
                sems.at[1, ba2]).start()

        kb = i - 1
        cb_ = w + kb * NW
        bb4 = lax.rem(kb + 4, 4)
        bb2 = lax.rem(kb + 2, 2)

        @pl.when(jnp.logical_and(kb >= 0, cb_ < NCHUNK))
        def _():
            pltpu.make_async_copy(
                dst_hbm.at[pl.ds(cb_ * C, C)], idx4.at[bb4],
                sems.at[0, bb4]).wait()
            pltpu.make_async_copy(
                ef_hbm.at[pl.ds(cb_ * C, C)], rows2.at[bb2],
                sems.at[1, bb2]).wait()
            pltpu.make_async_copy(
                rows2.at[bb2], acc.at[idx4.at[bb4]], sems.at[2, bb2]
            ).start(add=True)

    for b in range(2):
        pltpu.make_async_copy(
            rows2.at[b], acc.at[idx4.at[b]], sems.at[2, b]).wait()

    plsc.subcore_barrier()
    pltpu.sync_copy(acc.at[sl], eagg_hbm.at[pl.ds(c * NP + s * RPT, RPT)])


def _sc_cnt(dst_hbm, ones_hbm, z128_hbm, cnt_hbm, idx4, ones_v, sems, acc):
    c = lax.axis_index("c")
    s = lax.axis_index("s")
    w = s * 2 + c
    sl = pl.ds(s * RPT, RPT)

    pltpu.sync_copy(z128_hbm.at[sl], acc.at[sl])
    pltpu.sync_copy(ones_hbm, ones_v)
    plsc.subcore_barrier()

    @pl.loop(0, NLOOP + 1)
    def _(i):
        ka = i
        ca_ = w + ka * NW
        ba4 = lax.rem(ka, 4)

        ba2 = lax.rem(ka, 2)

        @pl.when(ca_ < NCHUNK)
        def _():
            @pl.when(ka >= 2)
            def _():
                pltpu.make_async_copy(
                    ones_v, acc.at[idx4.at[ba4]], sems.at[1, ba2]).wait()
            pltpu.make_async_copy(
                dst_hbm.at[pl.ds(ca_ * C, C)], idx4.at[ba4],
                sems.at[0, ba4]).start()

        kb = i - 1
        cb_ = w + kb * NW
        bb4 = lax.rem(kb + 4, 4)

        bb2 = lax.rem(kb + 2, 2)

        @pl.when(jnp.logical_and(kb >= 0, cb_ < NCHUNK))
        def _():
            pltpu.make_async_copy(
                dst_hbm.at[pl.ds(cb_ * C, C)], idx4.at[bb4],
                sems.at[0, bb4]).wait()
            pltpu.make_async_copy(
                ones_v, acc.at[idx4.at[bb4]], sems.at[1, bb2]
            ).start(add=True)

    for b in range(2):
        pltpu.make_async_copy(
            ones_v, acc.at[idx4.at[b]], sems.at[1, b]).wait()

    plsc.subcore_barrier()
    pltpu.sync_copy(acc.at[sl], cnt_hbm.at[pl.ds(c * NP + s * RPT, RPT)])


def _sc_gather(src_hbm, dst_hbm, h_hbm, z128_hbm, hagg_hbm,
               idx4, rows2, sems, acc):
    c = lax.axis_index("c")
    s = lax.axis_index("s")
    w = s * 2 + c
    sl = pl.ds(s * RPT, RPT)

    pltpu.sync_copy(z128_hbm.at[sl], acc.at[sl])
    plsc.subcore_barrier()

    @pl.loop(0, NLOOP + 2)
    def _(i):
        ka = i
        ca_ = w + ka * NW
        ba4 = lax.rem(ka, 4)

        @pl.when(ca_ < NCHUNK)
        def _():
            pltpu.make_async_copy(
                src_hbm.at[pl.ds(ca_ * C, C)], idx4.at[ba4, 0],
                sems.at[0, ba4]).start()
            pltpu.make_async_copy(
                dst_hbm.at[pl.ds(ca_ * C, C)], idx4.at[ba4, 1],
                sems.at[0, ba4]).start()

        kb = i - 1
        cb_ = w + kb * NW
        bb4 = lax.rem(kb + 4, 4)
        bb2 = lax.rem(kb + 2, 2)

        @pl.when(jnp.logical_and(kb >= 0, cb_ < NCHUNK))
        def _():
            @pl.when(kb >= 2)
            def _():
                pltpu.make_async_copy(
                    rows2.at[bb2], acc.at[idx4.at[bb4, 1]],
                    sems.at[2, bb2]).wait()
            pltpu.make_async_copy(
                src_hbm.at[pl.ds(cb_ * C, C)], idx4.at[bb4, 0],
                sems.at[0, bb4]).wait()
            pltpu.make_async_copy(
                dst_hbm.at[pl.ds(cb_ * C, C)], idx4.at[bb4, 1],
                sems.at[0, bb4]).wait()
            pltpu.make_async_copy(
                h_hbm.at[idx4.at[bb4, 0]], rows2.at[bb2],
                sems.at[1, bb2]).start()

        kc = i - 2
        cc_ = w + kc * NW
        bc4 = lax.rem(kc + 4, 4)
        bc2 = lax.rem(kc + 2, 2)

        @pl.when(jnp.logical_and(kc >= 0, cc_ < NCHUNK))
        def _():
            pltpu.make_async_copy(
                h_hbm.at[idx4.at[bc4, 0]], rows2.at[bc2],
                sems.at[1, bc2]).wait()
            pltpu.make_async_copy(
                rows2.at[bc2], acc.at[idx4.at[bc4, 1]], sems.at[2, bc2]
            ).start(add=True)

    for b in range(2):
        pltpu.make_async_copy(
            rows2.at[b], acc.at[idx4.at[b, 1]], sems.at[2, b]).wait()

    plsc.subcore_barrier()
    pltpu.sync_copy(acc.at[sl], hagg_hbm.at[pl.ds(c * NP + s * RPT, RPT)])


BLK = 1024
NB = NP // BLK

_DOT = functools.partial(
    lax.dot_general,
    dimension_numbers=(((1,), (0,)), ((), ())),
    preferred_element_type=jnp.float32,
    precision=lax.Precision.HIGHEST,
)


def _tc_mm0(ea_ref, eb_ref, ha_ref, hb_ref, w_ref, b_ref, out_ref):
    agg = (ea_ref[...] + eb_ref[...]) + (ha_ref[...] + hb_ref[...])
    out_ref[...] = _DOT(agg, w_ref[...]) + b_ref[...]


def _tc_fin0(mm_ref, ca_ref, cb_ref, nf_ref, out_ref):
    cnt = ca_ref[:, 0:1] + cb_ref[:, 0:1]
    nrm = lax.rsqrt(jnp.maximum(cnt, 1.0))
    out_ref[...] = jnp.maximum(mm_ref[...] * nrm + nf_ref[...], 0.0)


def _tc_layer1(ea_ref, eb_ref, ha_ref, hb_ref, ca_ref, cb_ref,
               w_ref, b_ref, out_ref):
    agg = (ea_ref[...] + eb_ref[...]) + (ha_ref[...] + hb_ref[...])
    cnt = ca_ref[:, 0:1] + cb_ref[:, 0:1]
    nrm = lax.rsqrt(jnp.maximum(cnt, 1.0))
    out_ref[...] = (_DOT(agg, w_ref[...]) + b_ref[...]) * nrm


def kernel(node_fea, edge_fea, edge_index, W0, b0, W1, b1):
    src = edge_index[0]
    dst = edge_index[1]
    z128 = jnp.zeros((NP, D), jnp.float32)
    ones128 = jnp.ones((C, D), jnp.float32)
    nf_pad = jnp.concatenate(
        [node_fea, jnp.zeros((NP - N, D), jnp.float32)], axis=0)

    sc_eagg = pl.kernel(
        _sc_eagg,
        out_type=jax.ShapeDtypeStruct((2 * NP, D), jnp.float32),
        mesh=_MESH,
        scratch_types=[
            pltpu.VMEM((4, C), jnp.int32),
            pltpu.VMEM((2, C, D), jnp.float32),
            pltpu.SemaphoreType.DMA((3, 4)),
            pltpu.VMEM_SHARED((NP, D), jnp.float32),
        ],
    )
    eagg2 = sc_eagg(dst, edge_fea, z128)

    sc_cnt = pl.kernel(
        _sc_cnt,
        out_type=jax.ShapeDtypeStruct((2 * NP, D), jnp.float32),
        mesh=_MESH,
        scratch_types=[
            pltpu.VMEM((4, C), jnp.int32),
            pltpu.VMEM((C, D), jnp.float32),
            pltpu.SemaphoreType.DMA((2, 4)),
            pltpu.VMEM_SHARED((NP, D), jnp.float32),
        ],
    )
    cnt2 = sc_cnt(dst, ones128, z128)

    sc_gather = pl.kernel(
        _sc_gather,
        out_type=jax.ShapeDtypeStruct((2 * NP, D), jnp.float32),
        mesh=_MESH,
        scratch_types=[
            pltpu.VMEM((4, 2, C), jnp.int32),
            pltpu.VMEM((2, C, D), jnp.float32),
            pltpu.SemaphoreType.DMA((3, 4)),
            pltpu.VMEM_SHARED((NP, D), jnp.float32),
        ],
    )
    h0agg2 = sc_gather(src, dst, nf_pad, z128)

    pa = pl.BlockSpec((BLK, D), lambda i: (i, 0))
    pb = pl.BlockSpec((BLK, D), lambda i: (i + NB, 0))
    wspec = pl.BlockSpec((D, D), lambda i: (0, 0))
    bspec = pl.BlockSpec((1, D), lambda i: (0, 0))

    mm0 = pl.pallas_call(
        _tc_mm0,
        grid=(NB,),
        in_specs=[pa, pb, pa, pb, wspec, bspec],
        out_specs=pa,
        out_shape=jax.ShapeDtypeStruct((NP, D), jnp.float32),
    )(eagg2, eagg2, h0agg2, h0agg2, W0, b0.reshape(1, D))

    h1 = pl.pallas_call(
        _tc_fin0,
        grid=(NB,),
        in_specs=[pa, pa, pb, pa],
        out_specs=pa,
        out_shape=jax.ShapeDtypeStruct((NP, D), jnp.float32),
    )(mm0, cnt2, cnt2, nf_pad)

    h1agg2 = sc_gather(src, dst, h1, z128)

    out = pl.pallas_call(
        _tc_layer1,
        grid=(NB,),
        in_specs=[pa, pb, pa, pb, pa, pb, wspec, bspec],
        out_specs=pa,
        out_shape=jax.ShapeDtypeStruct((NP, D), jnp.float32),
    )(eagg2, eagg2, h1agg2, h1agg2, cnt2, cnt2, W1, b1.reshape(1, D))

    return out[:N]


# speedup vs baseline: 1.0057x; 1.0057x over previous
import functools

import jax
import jax.numpy as jnp
from jax import lax
from jax.experimental import pallas as pl
from jax.experimental.pallas import tpu as pltpu
from jax.experimental.pallas import tpu_sc as plsc

N = 10000
NP = 10240
D = 128
E = 320000
C = 128
NCHUNK = E // C
NT = 16
NW = 32
RPT = NP // NT
KBASE = NCHUNK // NW
KEXTRA = NCHUNK - KBASE * NW
KMAX = KBASE + 1
KW = 88
NBUF = 4
NLOOP = (NCHUNK + NW - 1) // NW

_MESH = plsc.VectorSubcoreMesh(core_axis_name="c", subcore_axis_name="s")


def _sc_eagg(dst_hbm, ef_hbm, z128_hbm, ea0_hbm, ea1_hbm, idx4, rows2, sems, acc):
    c = lax.axis_index("c")
    s = lax.axis_index("s")
    w = s * 2 + c
    sl = pl.ds(s * RPT, RPT)

    pltpu.sync_copy(z128_hbm.at[sl], acc.at[sl])
    plsc.subcore_barrier()

    @pl.loop(0, NLOOP + 1)
    def _(i):
        ka = i
        ca_ = w + ka * NW
        ba4 = lax.rem(ka, 4)
        ba2 = lax.rem(ka, 2)

        @pl.when(ca_ < NCHUNK)
        def _():
            @pl.when(ka >= 2)
            def _():
                pltpu.make_async_copy(
                    rows2.at[ba2], acc.at[idx4.at[ba4]], sems.at[2, ba2]).wait()
            pltpu.make_async_copy(
                dst_hbm.at[pl.ds(ca_ * C, C)], idx4.at[ba4],
                sems.at[0, ba4]).start()
            pltpu.make_async_copy(
                ef_hbm.at[pl.ds(ca_ * C, C)], rows2.at[ba2],
                sems.at[1, ba2]).start()

        kb = i - 1
        cb_ = w + kb * NW
        bb4 = lax.rem(kb + 4, 4)
        bb2 = lax.rem(kb + 2, 2)

        @pl.when(jnp.logical_and(kb >= 0, cb_ < NCHUNK))
        def _():
            pltpu.make_async_copy(
                dst_hbm.at[pl.ds(cb_ * C, C)], idx4.at[bb4],
                sems.at[0, bb4]).wait()
            pltpu.make_async_copy(
                ef_hbm.at[pl.ds(cb_ * C, C)], rows2.at[bb2],
                sems.at[1, bb2]).wait()
            pltpu.make_async_copy(
                rows2.at[bb2], acc.at[idx4.at[bb4]], sems.at[2, bb2]
            ).start(add=True)

    for b in range(2):
        pltpu.make_async_copy(
            rows2.at[b], acc.at[idx4.at[b]], sems.at[2, b]).wait()

    plsc.subcore_barrier()

    @pl.when(c == 0)
    def _():
        pltpu.sync_copy(acc.at[sl], ea0_hbm.at[sl])

    @pl.when(c == 1)
    def _():
        pltpu.sync_copy(acc.at[sl], ea1_hbm.at[sl])


def _sc_cnt(dst_hbm, ones_hbm, z128_hbm, cn0_hbm, cn1_hbm, idx4, ones_v, sems, acc):
    c = lax.axis_index("c")
    s = lax.axis_index("s")
    w = s * 2 + c
    sl = pl.ds(s * RPT, RPT)

    pltpu.sync_copy(z128_hbm.at[sl], acc.at[sl])
    pltpu.sync_copy(ones_hbm, ones_v)
    plsc.subcore_barrier()

    @pl.loop(0, NLOOP + 1)
    def _(i):
        ka = i
        ca_ = w + ka * NW
        ba4 = lax.rem(ka, 4)

        ba2 = lax.rem(ka, 2)

        @pl.when(ca_ < NCHUNK)
        def _():
            @pl.when(ka >= 2)
            def _():
                pltpu.make_async_copy(
                    ones_v, acc.at[idx4.at[ba4]], sems.at[1, ba2]).wait()
            pltpu.make_async_copy(
                dst_hbm.at[pl.ds(ca_ * C, C)], idx4.at[ba4],
                sems.at[0, ba4]).start()

        kb = i - 1
        cb_ = w + kb * NW
        bb4 = lax.rem(kb + 4, 4)

        bb2 = lax.rem(kb + 2, 2)

        @pl.when(jnp.logical_and(kb >= 0, cb_ < NCHUNK))
        def _():
            pltpu.make_async_copy(
                dst_hbm.at[pl.ds(cb_ * C, C)], idx4.at[bb4],
                sems.at[0, bb4]).wait()
            pltpu.make_async_copy(
                ones_v, acc.at[idx4.at[bb4]], sems.at[1, bb2]
            ).start(add=True)

    for b in range(2):
        pltpu.make_async_copy(
            ones_v, acc.at[idx4.at[b]], sems.at[1, b]).wait()

    plsc.subcore_barrier()

    @pl.when(c == 0)
    def _():
        pltpu.sync_copy(acc.at[sl], cn0_hbm.at[sl])

    @pl.when(c == 1)
    def _():
        pltpu.sync_copy(acc.at[sl], cn1_hbm.at[sl])


def _sc_gather(src_hbm, dst_hbm, h_hbm, z128_hbm, ha0_hbm, ha1_hbm,
               idx4, rows2, sems, acc):
    c = lax.axis_index("c")
    s = lax.axis_index("s")
    w = s * 2 + c
    sl = pl.ds(s * RPT, RPT)

    pltpu.sync_copy(z128_hbm.at[sl], acc.at[sl])
    plsc.subcore_barrier()

    @pl.loop(0, NLOOP + 2)
    def _(i):
        ka = i
        ca_ = w + ka * NW
        ba4 = lax.rem(ka, 4)

        @pl.when(ca_ < NCHUNK)
        def _():
            pltpu.make_async_copy(
                src_hbm.at[pl.ds(ca_ * C, C)], idx4.at[ba4, 0],
                sems.at[0, ba4]).start()
            pltpu.make_async_copy(
                dst_hbm.at[pl.ds(ca_ * C, C)], idx4.at[ba4, 1],
                sems.at[0, ba4]).start()

        kb = i - 1
        cb_ = w + kb * NW
        bb4 = lax.rem(kb + 4, 4)
        bb2 = lax.rem(kb + 2, 2)

        @pl.when(jnp.logical_and(kb >= 0, cb_ < NCHUNK))
        def _():
            @pl.when(kb >= 2)
            def _():
                pltpu.make_async_copy(
                    rows2.at[bb2], acc.at[idx4.at[bb4, 1]],
                    sems.at[2, bb2]).wait()
            pltpu.make_async_copy(
                src_hbm.at[pl.ds(cb_ * C, C)], idx4.at[bb4, 0],
                sems.at[0, bb4]).wait()
            pltpu.make_async_copy(
                dst_hbm.at[pl.ds(cb_ * C, C)], idx4.at[bb4, 1],
                sems.at[0, bb4]).wait()
            pltpu.make_async_copy(
                h_hbm.at[idx4.at[bb4, 0]], rows2.at[bb2],
                sems.at[1, bb2]).start()

        kc = i - 2
        cc_ = w + kc * NW
        bc4 = lax.rem(kc + 4, 4)
        bc2 = lax.rem(kc + 2, 2)

        @pl.when(jnp.logical_and(kc >= 0, cc_ < NCHUNK))
        def _():
            pltpu.make_async_copy(
                h_hbm.at[idx4.at[bc4, 0]], rows2.at[bc2],
                sems.at[1, bc2]).wait()
            pltpu.make_async_copy(
                rows2.at[bc2], acc.at[idx4.at[bc4, 1]], sems.at[2, bc2]
            ).start(add=True)

    for b in range(2):
        pltpu.make_async_copy(
            rows2.at[b], acc.at[idx4.at[b, 1]], sems.at[2, b]).wait()

    plsc.subcore_barrier()

    @pl.when(c == 0)
    def _():
        pltpu.sync_copy(acc.at[sl], ha0_hbm.at[sl])

    @pl.when(c == 1)
    def _():
        pltpu.sync_copy(acc.at[sl], ha1_hbm.at[sl])


BLK = 1000
NB = N // BLK

_DOT = functools.partial(
    lax.dot_general,
    dimension_numbers=(((1,), (0,)), ((), ())),
    preferred_element_type=jnp.float32,
    precision=lax.Precision.HIGHEST,
)


def _tc_layer0(ea_ref, eb_ref, ha_ref, hb_ref, ca_ref, cb_ref, nf_ref,
               w_ref, b_ref, out_ref):
    agg = (ea_ref[...] + eb_ref[...]) + (ha_ref[...] + hb_ref[...])
    cnt = ca_ref[:, 0:1] + cb_ref[:, 0:1]
    nrm = lax.rsqrt(jnp.maximum(cnt, 1.0))
    rst = (_DOT(agg, w_ref[...]) + b_ref[...]) * nrm + nf_ref[...]
    out_ref[...] = jnp.maximum(rst, 0.0)


def _tc_layer1(ea_ref, eb_ref, ha_ref, hb_ref, ca_ref, cb_ref,
               w_ref, b_ref, out_ref):
    agg = (ea_ref[...] + eb_ref[...]) + (ha_ref[...] + hb_ref[...])
    cnt = ca_ref[:, 0:1] + cb_ref[:, 0:1]
    nrm = lax.rsqrt(jnp.maximum(cnt, 1.0))
    out_ref[...] = (_DOT(agg, w_ref[...]) + b_ref[...]) * nrm


def kernel(node_fea, edge_fea, edge_index, W0, b0, W1, b1):
    src = edge_index[0]
    dst = edge_index[1]
    z128 = jnp.zeros((NP, D), jnp.float32)
    ones128 = jnp.ones((C, D), jnp.float32)

    out2 = lambda: (jax.ShapeDtypeStruct((NP, D), jnp.float32),
                    jax.ShapeDtypeStruct((NP, D), jnp.float32))

    sc_eagg = pl.kernel(
        _sc_eagg,
        out_type=out2(),
        mesh=_MESH,
        scratch_types=[
            pltpu.VMEM((4, C), jnp.int32),
            pltpu.VMEM((2, C, D), jnp.float32),
            pltpu.SemaphoreType.DMA((3, 4)),
            pltpu.VMEM_SHARED((NP, D), jnp.float32),
        ],
    )
    ea0, ea1 = sc_eagg(dst, edge_fea, z128)

    sc_cnt = pl.kernel(
        _sc_cnt,
        out_type=out2(),
        mesh=_MESH,
        scratch_types=[
            pltpu.VMEM((4, C), jnp.int32),
            pltpu.VMEM((C, D), jnp.float32),
            pltpu.SemaphoreType.DMA((2, 4)),
            pltpu.VMEM_SHARED((NP, D), jnp.float32),
        ],
    )
    cn0, cn1 = sc_cnt(dst, ones128, z128)

    sc_gather = pl.kernel(
        _sc_gather,
        out_type=out2(),
        mesh=_MESH,
        scratch_types=[
            pltpu.VMEM((4, 2, C), jnp.int32),
            pltpu.VMEM((2, C, D), jnp.float32),
            pltpu.SemaphoreType.DMA((3, 4)),
            pltpu.VMEM_SHARED((NP, D), jnp.float32),
        ],
    )
    ha0, ha1 = sc_gather(src, dst, node_fea, z128)

    pa = pl.BlockSpec((BLK, D), lambda i: (i, 0))
    wspec = pl.BlockSpec((D, D), lambda i: (0, 0))
    bspec = pl.BlockSpec((1, D), lambda i: (0, 0))

    h1 = pl.pallas_call(
        _tc_layer0,
        grid=(NB,),
        in_specs=[pa, pa, pa, pa, pa, pa, pa, wspec, bspec],
        out_specs=pa,
        out_shape=jax.ShapeDtypeStruct((N, D), jnp.float32),
    )(ea0, ea1, ha0, ha1, cn0, cn1, node_fea, W0, b0.reshape(1, D))

    hb0, hb1 = sc_gather(src, dst, h1, z128)

    out = pl.pallas_call(
        _tc_layer1,
        grid=(NB,),
        in_specs=[pa, pa, pa, pa, pa, pa, wspec, bspec],
        out_specs=pa,
        out_shape=jax.ShapeDtypeStruct((N, D), jnp.float32),
    )(ea0, ea1, hb0, hb1, cn0, cn1, W1, b1.reshape(1, D))

    return out
